# baseline (speedup 1.0000x reference)
import jax
import jax.numpy as jnp
from jax import lax
from jax.experimental import pallas as pl
from jax.experimental.pallas import tpu as pltpu
from jax.experimental.pallas import tpu_sc as plsc

_BLOCK_R = 1024
_FIX_R = 8


def _scs_gather_body(mid_hbm, tab_hbm, row_hbm, mid_smem):
    pltpu.sync_copy(mid_hbm, mid_smem)
    m = mid_smem[0]
    pltpu.sync_copy(tab_hbm.at[pl.ds(m, 1)], row_hbm)


def _sc_gather(mid, mod_emb_table):
    D = mod_emb_table.shape[1]
    mesh = plsc.ScalarSubcoreMesh(axis_name="c", num_cores=1)
    return pl.kernel(
        _scs_gather_body,
        mesh=mesh,
        out_type=jax.ShapeDtypeStruct((1, D), mod_emb_table.dtype),
        scratch_types=[
            pltpu.SMEM((1,), jnp.int32),
        ],
    )(mid, mod_emb_table)


def _tc_bulk_body(mid_ref, x_ref, tab_ref, o_ref):
    o_ref[...] = x_ref[...] + tab_ref[0]


def _tc_fix_body(y_ref, x_ref, row_ref, o_ref):
    o_ref[...] = x_ref[...] + row_ref[...]


def kernel(x, mod_emb_table, modality_id):
    B, S, D = x.shape
    R = B * S
    M = mod_emb_table.shape[0]
    xf = x.reshape(R, D)
    tab3 = mod_emb_table.reshape(M, 1, D)
    mid = jnp.asarray(modality_id, jnp.int32).reshape(1)

    row = _sc_gather(mid, mod_emb_table)

    y = pl.pallas_call(
        _tc_bulk_body,
        grid_spec=pltpu.PrefetchScalarGridSpec(
            num_scalar_prefetch=1,
            grid=(R // _BLOCK_R,),
            in_specs=[
                pl.BlockSpec((_BLOCK_R, D), lambda i, mid: (i, 0)),
                pl.BlockSpec((1, 1, D), lambda i, mid: (mid[0], 0, 0)),
            ],
            out_specs=pl.BlockSpec((_BLOCK_R, D), lambda i, mid: (i, 0)),
        ),
        out_shape=jax.ShapeDtypeStruct((R, D), x.dtype),
    )(mid, xf, tab3)

    out = pl.pallas_call(
        _tc_fix_body,
        grid=(1,),
        in_specs=[
            pl.BlockSpec((_FIX_R, D), lambda i: (0, 0)),
            pl.BlockSpec((_FIX_R, D), lambda i: (0, 0)),
            pl.BlockSpec((1, D), lambda i: (0, 0)),
        ],
        out_specs=pl.BlockSpec((_FIX_R, D), lambda i: (0, 0)),
        out_shape=jax.ShapeDtypeStruct((R, D), x.dtype),
        input_output_aliases={0: 0},
    )(y, xf, row)
    return out.reshape(B, S, D)


# speedup vs baseline: 1.0024x; 1.0024x over previous
import jax
import jax.numpy as jnp
from jax.experimental import pallas as pl
from jax.experimental.pallas import tpu as pltpu
from jax.experimental.pallas import tpu_sc as plsc

_BLOCK_R = 1024


def _scs_gather_body(mid_hbm, tab_hbm, row_hbm, mid_smem):
    pltpu.sync_copy(mid_hbm, mid_smem)
    m = mid_smem[0]
    pltpu.sync_copy(tab_hbm.at[pl.ds(m, 1)], row_hbm)


def _sc_gather(mid, mod_emb_table):
    D = mod_emb_table.shape[1]
    mesh = plsc.ScalarSubcoreMesh(axis_name="c", num_cores=1)
    return pl.kernel(
        _scs_gather_body,
        mesh=mesh,
        out_type=jax.ShapeDtypeStruct((1, D), mod_emb_table.dtype),
        scratch_types=[
            pltpu.SMEM((1,), jnp.int32),
        ],
    )(mid, mod_emb_table)


def _tc_add_body(x_ref, row_ref, o_ref):
    o_ref[...] = x_ref[...] + row_ref[...]


def kernel(x, mod_emb_table, modality_id):
    B, S, D = x.shape
    R = B * S
    xf = x.reshape(R, D)
    mid = jnp.asarray(modality_id, jnp.int32).reshape(1)
    row = _sc_gather(mid, mod_emb_table)
    out = pl.pallas_call(
        _tc_add_body,
        grid=(R // _BLOCK_R,),
        in_specs=[
            pl.BlockSpec((_BLOCK_R, D), lambda i: (i, 0)),
            pl.BlockSpec((1, D), lambda i: (0, 0)),
        ],
        out_specs=pl.BlockSpec((_BLOCK_R, D), lambda i: (i, 0)),
        out_shape=jax.ShapeDtypeStruct((R, D), x.dtype),
    )(xf, row)
    return out.reshape(B, S, D)
